# baseline (speedup 1.0000x reference)
import jax
import jax.numpy as jnp
from jax.experimental import pallas as pl
from jax.experimental.pallas import tpu as pltpu

N_ROWS = 1000000
BLOCK_ROWS = 20000


def _concat_body(lt_ref, x_ref, out_ref):
    half = BLOCK_ROWS // 2
    even = pl.Slice(0, half, 2)
    odd = pl.Slice(1, half, 2)
    out_ref[even, :] = jnp.concatenate([lt_ref[:, 0:64], x_ref[:, 0:64]], axis=1)
    out_ref[odd, :] = jnp.concatenate([lt_ref[:, 64:128], x_ref[:, 64:128]], axis=1)


def kernel(x, adj, lt):
    del adj
    n = lt.shape[0]
    lt2 = lt.reshape(n // 2, 128)
    x2 = x.reshape(n // 2, 128)
    grid = (n // BLOCK_ROWS,)
    return pl.pallas_call(
        _concat_body,
        grid=grid,
        in_specs=[
            pl.BlockSpec((BLOCK_ROWS // 2, 128), lambda i: (i, 0)),
            pl.BlockSpec((BLOCK_ROWS // 2, 128), lambda i: (i, 0)),
        ],
        out_specs=pl.BlockSpec((BLOCK_ROWS, 128), lambda i: (i, 0)),
        out_shape=jax.ShapeDtypeStruct((n, 128), jnp.float32),
        compiler_params=pltpu.CompilerParams(allow_input_fusion=[True, True]),
    )(lt2, x2)
